# baseline (speedup 1.0000x reference)
import functools

import jax
import jax.numpy as jnp
from jax import lax
from jax.experimental import pallas as pl
from jax.experimental.pallas import tpu as pltpu
from jax.experimental.pallas import tpu_sc as plsc

_PRUNE_RATIO = 0.5

_NC = 2
_NS = 16
_NW = _NC * _NS
_L = 16

_NBUCKETS = 1024


_UNROLL = 4


def _hist_body(shift, masked, n_per_tile, chunk, scores_hbm, lo_hbm, out_hbm,
               buf, lo_v, hist_l, hist_1):
    c = lax.axis_index("c")
    s = lax.axis_index("s")
    wid = s * _NC + c

    pltpu.sync_copy(lo_hbm, lo_v)
    lo = lo_v[...]
    hi = lo + (_NBUCKETS << shift)
    lane = lax.iota(jnp.int32, _L)
    ones = jnp.ones((_L,), jnp.int32)
    zeros = jnp.zeros((_L,), jnp.int32)

    def zero_body(i, carry):
        hist_l[pl.ds(i * _L, _L)] = zeros
        return carry

    lax.fori_loop(0, _NBUCKETS * _UNROLL, zero_body, 0)

    base = wid * n_per_tile
    nchunks = n_per_tile // chunk

    def chunk_body(ci, carry):
        pltpu.sync_copy(scores_hbm.at[pl.ds(base + ci * chunk, chunk)], buf)

        def vec_body(i, inner):
            for u in range(_UNROLL):
                x = buf[pl.ds((i * _UNROLL + u) * _L, _L)]
                bits = x & jnp.int32(0x7FFFFFFF)
                b = (bits - lo) >> shift if shift else (bits - lo)
                b = jnp.clip(b, 0, _NBUCKETS - 1)
                addr = (u * _NBUCKETS + b) * _L + lane
                if masked:
                    in_rng = (bits >= lo) & (bits < hi)
                    plsc.addupdate_scatter(hist_l, [addr], ones, mask=in_rng)
                else:
                    plsc.addupdate_scatter(hist_l, [addr], ones)
            return inner

        lax.fori_loop(0, chunk // (_L * _UNROLL), vec_body, 0)
        return carry

    lax.fori_loop(0, nchunks, chunk_body, 0)

    def red_body(g, carry):
        bvec = lax.iota(jnp.int32, _L) + g * _L
        acc = jnp.zeros((_L,), jnp.int32)
        for u in range(_UNROLL):
            for l in range(_L):
                acc = acc + plsc.load_gather(
                    hist_l, [(u * _NBUCKETS + bvec) * _L + l])
        hist_1[pl.ds(g * _L, _L)] = acc
        return carry

    lax.fori_loop(0, _NBUCKETS // _L, red_body, 0)

    pltpu.sync_copy(hist_1, out_hbm.at[wid])


def _hist_pass(shift, masked, n_total, chunk):
    n_per_tile = n_total // _NW
    mesh = plsc.VectorSubcoreMesh(core_axis_name="c", subcore_axis_name="s")
    return pl.kernel(
        functools.partial(_hist_body, shift, masked, n_per_tile, chunk),
        out_type=jax.ShapeDtypeStruct((_NW, _NBUCKETS), jnp.int32),
        mesh=mesh,
        compiler_params=pltpu.CompilerParams(needs_layout_passes=False),
        scratch_types=[
            pltpu.VMEM((chunk,), jnp.int32),
            pltpu.VMEM((_L,), jnp.int32),
            pltpu.VMEM((_NBUCKETS * _L * _UNROLL,), jnp.int32),
            pltpu.VMEM((_NBUCKETS,), jnp.int32),
        ],
    )


def _mask_body(t_ref, w_ref, s_ref, o_ref):
    t = t_ref[0]
    sv = s_ref[...]
    a = jnp.abs(sv)
    bits = lax.bitcast_convert_type(sv, jnp.int32) & jnp.int32(0x7FFFFFFF)
    binary = jnp.where(bits >= t, jnp.float32(1.0), jnp.float32(0.0))
    m = a + (binary - a)
    o_ref[...] = w_ref[...] * m


def _apply_mask(weight, scores, t_bits):
    rows, dim = weight.shape
    block_rows = 2000
    grid = rows // block_rows
    return pl.pallas_call(
        _mask_body,
        grid=(grid,),
        in_specs=[
            pl.BlockSpec(memory_space=pltpu.SMEM),
            pl.BlockSpec((block_rows, dim), lambda i: (i, 0)),
            pl.BlockSpec((block_rows, dim), lambda i: (i, 0)),
        ],
        out_specs=pl.BlockSpec((block_rows, dim), lambda i: (i, 0)),
        out_shape=jax.ShapeDtypeStruct((rows, dim), jnp.float32),
    )(jnp.full((1,), t_bits, jnp.int32), weight, scores)


def _gather_body(b_per_w, gchunk, dim, tbl_hbm, idx_hbm, out_hbm,
                 idx_c, rows_v, sem):
    c = lax.axis_index("c")
    s = lax.axis_index("s")
    wid = s * _NC + c
    base = wid * b_per_w

    def chunk_body(ci, carry):
        row0 = base + ci * gchunk
        pltpu.sync_copy(idx_hbm.at[pl.ds(row0, gchunk)], idx_c)
        pltpu.async_copy(tbl_hbm.at[idx_c], rows_v, sem).wait()
        pltpu.sync_copy(rows_v, out_hbm.at[pl.ds(row0, gchunk)])
        return carry

    lax.fori_loop(0, b_per_w // gchunk, chunk_body, 0)


def _gather(table, idx_flat):
    nrows, dim = table.shape
    n_idx = idx_flat.shape[0]
    b_per_w = n_idx // _NW
    gchunk = 800
    mesh = plsc.VectorSubcoreMesh(core_axis_name="c", subcore_axis_name="s")
    return pl.kernel(
        functools.partial(_gather_body, b_per_w, gchunk, dim),
        out_type=jax.ShapeDtypeStruct((n_idx, dim), jnp.float32),
        mesh=mesh,
        compiler_params=pltpu.CompilerParams(needs_layout_passes=False,
                                             use_tc_tiling_on_sc=False),
        scratch_types=[
            pltpu.VMEM((gchunk,), jnp.int32),
            pltpu.VMEM((gchunk, dim), jnp.float32),
            pltpu.SemaphoreType.DMA,
        ],
    )(table, idx_flat)


def kernel(input, weight, scores):
    n_total = weight.shape[0] * weight.shape[1]
    j = int((1.0 - _PRUNE_RATIO) * n_total)

    sbits = lax.bitcast_convert_type(scores.reshape(-1), jnp.int32)
    lo = jnp.zeros((_L,), jnp.int32)
    j_rem = jnp.int32(j)
    for shift in (20, 10, 0):
        part = _hist_pass(shift, shift != 20, n_total, 8000)(sbits, lo)
        h = jnp.sum(part, axis=0)
        cum = jnp.cumsum(h)
        b = jnp.minimum(jnp.sum((cum <= j_rem).astype(jnp.int32)),
                        jnp.int32(_NBUCKETS - 1))
        below = jnp.take(cum, b) - jnp.take(h, b)
        j_rem = j_rem - below
        lo = lo + (b << shift).astype(jnp.int32)

    t_bits = lo[0]
    masked = _apply_mask(weight, scores, t_bits)
    out = _gather(masked, input.reshape(-1))
    return out.reshape(input.shape + (weight.shape[1],))


# speedup vs baseline: 1.0467x; 1.0467x over previous
import functools

import jax
import jax.numpy as jnp
from jax import lax
from jax.experimental import pallas as pl
from jax.experimental.pallas import tpu as pltpu
from jax.experimental.pallas import tpu_sc as plsc

_PRUNE_RATIO = 0.5

_NC = 2
_NS = 16
_NW = _NC * _NS
_L = 16

_NBUCKETS = 1024


_UNROLL = 4


def _hist_body(shift, masked, n_per_tile, chunk, scores_hbm, lo_hbm, out_hbm,
               buf, lo_v, hist_1, *hists):
    c = lax.axis_index("c")
    s = lax.axis_index("s")
    wid = s * _NC + c

    pltpu.sync_copy(lo_hbm, lo_v)
    lo = lo_v[...]
    hi = lo + (_NBUCKETS << shift)
    lane = lax.iota(jnp.int32, _L)
    ones = jnp.ones((_L,), jnp.int32)
    zeros = jnp.zeros((_L,), jnp.int32)

    def zero_body(i, carry):
        for u in range(_UNROLL):
            hists[u][pl.ds(i * _L, _L)] = zeros
        return carry

    lax.fori_loop(0, _NBUCKETS, zero_body, 0)

    base = wid * n_per_tile
    nchunks = n_per_tile // chunk

    def chunk_body(ci, carry):
        pltpu.sync_copy(scores_hbm.at[pl.ds(base + ci * chunk, chunk)], buf)

        def vec_body(i, inner):
            for u in range(_UNROLL):
                x = buf[pl.ds((i * _UNROLL + u) * _L, _L)]
                bits = x & jnp.int32(0x7FFFFFFF)
                b = (bits - lo) >> shift if shift else (bits - lo)
                b = jnp.clip(b, 0, _NBUCKETS - 1)
                addr = b * _L + lane
                if masked:
                    in_rng = (bits >= lo) & (bits < hi)
                    plsc.addupdate_scatter(hists[u], [addr], ones, mask=in_rng)
                else:
                    plsc.addupdate_scatter(hists[u], [addr], ones)
            return inner

        lax.fori_loop(0, chunk // (_L * _UNROLL), vec_body, 0)
        return carry

    lax.fori_loop(0, nchunks, chunk_body, 0)

    def red_body(g, carry):
        bvec = lax.iota(jnp.int32, _L) + g * _L
        acc = jnp.zeros((_L,), jnp.int32)
        for u in range(_UNROLL):
            for l in range(_L):
                acc = acc + plsc.load_gather(hists[u], [bvec * _L + l])
        hist_1[pl.ds(g * _L, _L)] = acc
        return carry

    lax.fori_loop(0, _NBUCKETS // _L, red_body, 0)

    pltpu.sync_copy(hist_1, out_hbm.at[wid])


def _hist_pass(shift, masked, n_total, chunk):
    n_per_tile = n_total // _NW
    mesh = plsc.VectorSubcoreMesh(core_axis_name="c", subcore_axis_name="s")
    return pl.kernel(
        functools.partial(_hist_body, shift, masked, n_per_tile, chunk),
        out_type=jax.ShapeDtypeStruct((_NW, _NBUCKETS), jnp.int32),
        mesh=mesh,
        compiler_params=pltpu.CompilerParams(needs_layout_passes=False),
        scratch_types=[
            pltpu.VMEM((chunk,), jnp.int32),
            pltpu.VMEM((_L,), jnp.int32),
            pltpu.VMEM((_NBUCKETS,), jnp.int32),
        ] + [pltpu.VMEM((_NBUCKETS * _L,), jnp.int32)
             for _ in range(_UNROLL)],
    )


def _mask_body(t_ref, w_ref, s_ref, o_ref):
    t = t_ref[0]
    sv = s_ref[...]
    a = jnp.abs(sv)
    bits = lax.bitcast_convert_type(sv, jnp.int32) & jnp.int32(0x7FFFFFFF)
    binary = jnp.where(bits >= t, jnp.float32(1.0), jnp.float32(0.0))
    m = a + (binary - a)
    o_ref[...] = w_ref[...] * m


def _apply_mask(weight, scores, t_bits):
    rows, dim = weight.shape
    block_rows = 2000
    grid = rows // block_rows
    return pl.pallas_call(
        _mask_body,
        grid=(grid,),
        in_specs=[
            pl.BlockSpec(memory_space=pltpu.SMEM),
            pl.BlockSpec((block_rows, dim), lambda i: (i, 0)),
            pl.BlockSpec((block_rows, dim), lambda i: (i, 0)),
        ],
        out_specs=pl.BlockSpec((block_rows, dim), lambda i: (i, 0)),
        out_shape=jax.ShapeDtypeStruct((rows, dim), jnp.float32),
    )(jnp.full((1,), t_bits, jnp.int32), weight, scores)


def _gather_body(b_per_w, gchunk, dim, tbl_hbm, idx_hbm, out_hbm,
                 idx_c, rows_v, sem):
    c = lax.axis_index("c")
    s = lax.axis_index("s")
    wid = s * _NC + c
    base = wid * b_per_w

    def chunk_body(ci, carry):
        row0 = base + ci * gchunk
        pltpu.sync_copy(idx_hbm.at[pl.ds(row0, gchunk)], idx_c)
        pltpu.async_copy(tbl_hbm.at[idx_c], rows_v, sem).wait()
        pltpu.sync_copy(rows_v, out_hbm.at[pl.ds(row0, gchunk)])
        return carry

    lax.fori_loop(0, b_per_w // gchunk, chunk_body, 0)


def _gather(table, idx_flat):
    nrows, dim = table.shape
    n_idx = idx_flat.shape[0]
    b_per_w = n_idx // _NW
    gchunk = 800
    mesh = plsc.VectorSubcoreMesh(core_axis_name="c", subcore_axis_name="s")
    return pl.kernel(
        functools.partial(_gather_body, b_per_w, gchunk, dim),
        out_type=jax.ShapeDtypeStruct((n_idx, dim), jnp.float32),
        mesh=mesh,
        compiler_params=pltpu.CompilerParams(needs_layout_passes=False,
                                             use_tc_tiling_on_sc=False),
        scratch_types=[
            pltpu.VMEM((gchunk,), jnp.int32),
            pltpu.VMEM((gchunk, dim), jnp.float32),
            pltpu.SemaphoreType.DMA,
        ],
    )(table, idx_flat)


def kernel(input, weight, scores):
    n_total = weight.shape[0] * weight.shape[1]
    j = int((1.0 - _PRUNE_RATIO) * n_total)

    sbits = lax.bitcast_convert_type(scores.reshape(-1), jnp.int32)
    lo = jnp.zeros((_L,), jnp.int32)
    j_rem = jnp.int32(j)
    for shift in (20, 10, 0):
        part = _hist_pass(shift, shift != 20, n_total, 8000)(sbits, lo)
        h = jnp.sum(part, axis=0)
        cum = jnp.cumsum(h)
        b = jnp.minimum(jnp.sum((cum <= j_rem).astype(jnp.int32)),
                        jnp.int32(_NBUCKETS - 1))
        below = jnp.take(cum, b) - jnp.take(h, b)
        j_rem = j_rem - below
        lo = lo + (b << shift).astype(jnp.int32)

    t_bits = lo[0]
    masked = _apply_mask(weight, scores, t_bits)
    out = _gather(masked, input.reshape(-1))
    return out.reshape(input.shape + (weight.shape[1],))


# speedup vs baseline: 1.6386x; 1.5655x over previous
import functools

import jax
import jax.numpy as jnp
from jax import lax
from jax.experimental import pallas as pl
from jax.experimental.pallas import tpu as pltpu
from jax.experimental.pallas import tpu_sc as plsc

_PRUNE_RATIO = 0.5

_NC = 2
_NS = 16
_NW = _NC * _NS
_L = 16

_NBUCKETS = 1024


_UNROLL = 4


def _hist_body(shift, masked, n_per_tile, chunk, scores_hbm, lo_hbm, out_hbm,
               buf0, buf1, lo_v, hist_l, hist_1, sem0, sem1):
    c = lax.axis_index("c")
    s = lax.axis_index("s")
    wid = s * _NC + c

    pltpu.sync_copy(lo_hbm, lo_v)
    lo = lo_v[...]
    hi = lo + (_NBUCKETS << shift)
    lane = lax.iota(jnp.int32, _L)
    ones = jnp.ones((_L,), jnp.int32)
    zeros = jnp.zeros((_L,), jnp.int32)

    @plsc.parallel_loop(0, _NBUCKETS * _UNROLL)
    def _(i):
        hist_l[pl.ds(i * _L, _L)] = zeros

    base = wid * n_per_tile
    nchunks = n_per_tile // chunk
    bufs = (buf0, buf1)
    sems = (sem0, sem1)

    def start(ci):
        return pltpu.async_copy(
            scores_hbm.at[pl.ds(base + ci * chunk, chunk)],
            bufs[ci % 2], sems[ci % 2])

    handles = {0: start(0)}
    for ci in range(nchunks):
        handles.pop(ci).wait()
        if ci + 1 < nchunks:
            handles[ci + 1] = start(ci + 1)
        bufc = bufs[ci % 2]

        @plsc.parallel_loop(0, chunk // _L, 1, unroll=_UNROLL)
        def _(i):
            x = bufc[pl.ds(i * _L, _L)]
            bits = x & jnp.int32(0x7FFFFFFF)
            b = (bits - lo) >> shift if shift else (bits - lo)
            b = jnp.clip(b, 0, _NBUCKETS - 1)
            stripe = (i & (_UNROLL - 1)) * (_NBUCKETS * _L)
            addr = b * _L + lane + stripe
            if masked:
                in_rng = (bits >= lo) & (bits < hi)
                plsc.addupdate_scatter(hist_l, [addr], ones, mask=in_rng)
            else:
                plsc.addupdate_scatter(hist_l, [addr], ones)

    @plsc.parallel_loop(0, _NBUCKETS // _L)
    def _(g):
        bvec = lax.iota(jnp.int32, _L) + g * _L
        vals = [plsc.load_gather(hist_l, [(u * _NBUCKETS + bvec) * _L + l])
                for u in range(_UNROLL) for l in range(_L)]
        while len(vals) > 1:
            vals = ([vals[i] + vals[i + 1] for i in range(0, len(vals) - 1, 2)]
                    + ([vals[-1]] if len(vals) % 2 else []))
        hist_1[pl.ds(g * _L, _L)] = vals[0]

    pltpu.sync_copy(hist_1, out_hbm.at[wid])


def _hist_pass(shift, masked, n_total, chunk):
    n_per_tile = n_total // _NW
    mesh = plsc.VectorSubcoreMesh(core_axis_name="c", subcore_axis_name="s")
    return pl.kernel(
        functools.partial(_hist_body, shift, masked, n_per_tile, chunk),
        out_type=jax.ShapeDtypeStruct((_NW, _NBUCKETS), jnp.int32),
        mesh=mesh,
        compiler_params=pltpu.CompilerParams(needs_layout_passes=False),
        scratch_types=[
            pltpu.VMEM((chunk,), jnp.int32),
            pltpu.VMEM((chunk,), jnp.int32),
            pltpu.VMEM((_L,), jnp.int32),
            pltpu.VMEM((_NBUCKETS * _L * _UNROLL,), jnp.int32),
            pltpu.VMEM((_NBUCKETS,), jnp.int32),
            pltpu.SemaphoreType.DMA,
            pltpu.SemaphoreType.DMA,
        ],
    )


def _mask_body(t_ref, w_ref, s_ref, o_ref):
    t = t_ref[0]
    sv = s_ref[...]
    a = jnp.abs(sv)
    bits = lax.bitcast_convert_type(sv, jnp.int32) & jnp.int32(0x7FFFFFFF)
    binary = jnp.where(bits >= t, jnp.float32(1.0), jnp.float32(0.0))
    m = a + (binary - a)
    o_ref[...] = w_ref[...] * m


def _apply_mask(weight, scores, t_bits):
    rows, dim = weight.shape
    block_rows = 2000
    grid = rows // block_rows
    return pl.pallas_call(
        _mask_body,
        grid=(grid,),
        in_specs=[
            pl.BlockSpec(memory_space=pltpu.SMEM),
            pl.BlockSpec((block_rows, dim), lambda i: (i, 0)),
            pl.BlockSpec((block_rows, dim), lambda i: (i, 0)),
        ],
        out_specs=pl.BlockSpec((block_rows, dim), lambda i: (i, 0)),
        out_shape=jax.ShapeDtypeStruct((rows, dim), jnp.float32),
    )(jnp.full((1,), t_bits, jnp.int32), weight, scores)


def _gather_body(b_per_w, gchunk, dim, tbl_hbm, idx_hbm, out_hbm,
                 idx_c, rows_v, sem):
    c = lax.axis_index("c")
    s = lax.axis_index("s")
    wid = s * _NC + c
    base = wid * b_per_w

    def chunk_body(ci, carry):
        row0 = base + ci * gchunk
        pltpu.sync_copy(idx_hbm.at[pl.ds(row0, gchunk)], idx_c)
        pltpu.async_copy(tbl_hbm.at[idx_c], rows_v, sem).wait()
        pltpu.sync_copy(rows_v, out_hbm.at[pl.ds(row0, gchunk)])
        return carry

    lax.fori_loop(0, b_per_w // gchunk, chunk_body, 0)


def _gather(table, idx_flat):
    nrows, dim = table.shape
    n_idx = idx_flat.shape[0]
    b_per_w = n_idx // _NW
    gchunk = 800
    mesh = plsc.VectorSubcoreMesh(core_axis_name="c", subcore_axis_name="s")
    return pl.kernel(
        functools.partial(_gather_body, b_per_w, gchunk, dim),
        out_type=jax.ShapeDtypeStruct((n_idx, dim), jnp.float32),
        mesh=mesh,
        compiler_params=pltpu.CompilerParams(needs_layout_passes=False,
                                             use_tc_tiling_on_sc=False),
        scratch_types=[
            pltpu.VMEM((gchunk,), jnp.int32),
            pltpu.VMEM((gchunk, dim), jnp.float32),
            pltpu.SemaphoreType.DMA,
        ],
    )(table, idx_flat)


def kernel(input, weight, scores):
    n_total = weight.shape[0] * weight.shape[1]
    j = int((1.0 - _PRUNE_RATIO) * n_total)

    sbits = lax.bitcast_convert_type(scores.reshape(-1), jnp.int32)
    lo = jnp.zeros((_L,), jnp.int32)
    j_rem = jnp.int32(j)
    for shift in (20, 10, 0):
        part = _hist_pass(shift, shift != 20, n_total, 20000)(sbits, lo)
        h = jnp.sum(part, axis=0)
        cum = jnp.cumsum(h)
        b = jnp.minimum(jnp.sum((cum <= j_rem).astype(jnp.int32)),
                        jnp.int32(_NBUCKETS - 1))
        below = jnp.take(cum, b) - jnp.take(h, b)
        j_rem = j_rem - below
        lo = lo + (b << shift).astype(jnp.int32)

    t_bits = lo[0]
    masked = _apply_mask(weight, scores, t_bits)
    out = _gather(masked, input.reshape(-1))
    return out.reshape(input.shape + (weight.shape[1],))
